# Initial kernel scaffold; baseline (speedup 1.0000x reference)
import jax
import jax.numpy as jnp
from jax.experimental import pallas as pl


def kernel(x):
    raise NotImplementedError("write your pallas kernel here")


# speedup vs baseline: 6.8152x; 6.8152x over previous
import numpy as np
import jax
import jax.numpy as jnp
from jax.experimental import pallas as pl

_GROUPED = np.array(
    [0, 8, 16, 24, 32, 40, 48, 56, 1, 9, 17, 25, 33, 41, 49, 57,
     2, 10, 18, 26, 34, 42, 50, 58, 3, 11, 19, 27, 35, 43, 51, 59,
     4, 12, 20, 28, 36, 44, 52, 60, 5, 13, 21, 29, 37, 45, 53, 61,
     6, 14, 22, 30, 38, 46, 54, 62, 7, 15, 23, 31, 39, 47, 55, 63],
    dtype=np.int64)
_RESTORED = _GROUPED.copy()
_PERM = _GROUPED[_RESTORED]


def _contiguous_runs(perm):
    runs = []
    n = len(perm)
    i = 0
    while i < n:
        j = i + 1
        while j < n and perm[j] == perm[j - 1] + 1:
            j += 1
        runs.append((i, int(perm[i]), j - i))
        i = j
    return runs


_RUNS = _contiguous_runs(_PERM)


def _permute_body(x_ref, o_ref):
    for dst, src, ln in _RUNS:
        o_ref[:, dst:dst + ln, :] = x_ref[:, src:src + ln, :]


def kernel(x):
    b, n, c = x.shape
    bb = 64
    grid = (b // bb,)
    return pl.pallas_call(
        _permute_body,
        grid=grid,
        in_specs=[pl.BlockSpec((bb, n, c), lambda i: (i, 0, 0))],
        out_specs=pl.BlockSpec((bb, n, c), lambda i: (i, 0, 0)),
        out_shape=jax.ShapeDtypeStruct((b, n, c), x.dtype),
    )(x)


# speedup vs baseline: 6.8973x; 1.0120x over previous
import numpy as np
import jax
import jax.numpy as jnp
from jax.experimental import pallas as pl

_GROUPED = np.array(
    [0, 8, 16, 24, 32, 40, 48, 56, 1, 9, 17, 25, 33, 41, 49, 57,
     2, 10, 18, 26, 34, 42, 50, 58, 3, 11, 19, 27, 35, 43, 51, 59,
     4, 12, 20, 28, 36, 44, 52, 60, 5, 13, 21, 29, 37, 45, 53, 61,
     6, 14, 22, 30, 38, 46, 54, 62, 7, 15, 23, 31, 39, 47, 55, 63],
    dtype=np.int64)
_RESTORED = _GROUPED.copy()
_PERM = _GROUPED[_RESTORED]


def _contiguous_runs(perm):
    runs = []
    n = len(perm)
    i = 0
    while i < n:
        j = i + 1
        while j < n and perm[j] == perm[j - 1] + 1:
            j += 1
        runs.append((i, int(perm[i]), j - i))
        i = j
    return runs


_RUNS = _contiguous_runs(_PERM)


def _permute_body(x_ref, o_ref):
    for dst, src, ln in _RUNS:
        o_ref[:, dst:dst + ln, :] = x_ref[:, src:src + ln, :]


def kernel(x):
    b, n, c = x.shape
    bb = 128
    grid = (b // bb,)
    return pl.pallas_call(
        _permute_body,
        grid=grid,
        in_specs=[pl.BlockSpec((bb, n, c), lambda i: (i, 0, 0))],
        out_specs=pl.BlockSpec((bb, n, c), lambda i: (i, 0, 0)),
        out_shape=jax.ShapeDtypeStruct((b, n, c), x.dtype),
    )(x)
